# baseline (speedup 1.0000x reference)
import functools

import jax
import jax.numpy as jnp
from jax import lax
from jax.experimental import pallas as pl
from jax.experimental.pallas import tpu as pltpu

_ROT0 = (13, 15, 26, 6)
_ROT1 = (17, 29, 16, 24)
_TINY = float.fromhex("0x1p-126")
_NEG_BIG = -3.0e38
_SUB = 256


def _log2(x):
    return jnp.log2(x)


def _rotl(x, d):
    return lax.shift_left(x, jnp.uint32(d)) | lax.shift_right_logical(
        x, jnp.uint32(32 - d)
    )


def _threefry_bits(i):
    ks = (0, 42, 42 ^ 0x1BD11BDA)
    x1 = i + jnp.uint32(42)
    x0 = x1
    x1 = _rotl(x1, _ROT0[0]) ^ x0
    for r in _ROT0[1:]:
        x0 = x0 + x1
        x1 = _rotl(x1, r)
        x1 = x1 ^ x0
    x0 = x0 + jnp.uint32(ks[1] & 0xFFFFFFFF)
    x1 = x1 + jnp.uint32((ks[2] + 1) & 0xFFFFFFFF)
    for g in range(1, 5):
        rots = _ROT0 if g % 2 == 0 else _ROT1
        for r in rots:
            x0 = x0 + x1
            x1 = _rotl(x1, r)
            x1 = x1 ^ x0
        x0 = x0 + jnp.uint32(ks[(g + 1) % 3] & 0xFFFFFFFF)
        x1 = x1 + jnp.uint32((ks[(g + 2) % 3] + g + 1) & 0xFFFFFFFF)
    return x0 ^ x1


def _gumbel_from_bits(bits):
    fb = lax.shift_right_logical(bits, jnp.uint32(9)) | jnp.uint32(0x3F800000)
    u = lax.bitcast_convert_type(fb, jnp.float32) - jnp.float32(1.0)
    uu = u + _TINY
    nln2 = jnp.float32(-0.6931471805599453)
    m1 = _log2(uu) * nln2
    return _log2(m1) * nln2


def _body(x_ref, rl_ref, o_ref, vb_ref, ib_ref, *, ncols, chunk, nrows, grid):
    j = pl.program_id(0)
    nsub = chunk // _SUB

    @pl.when(j == 0)
    def _init():
        vb_ref[...] = jnp.full((nrows, _SUB), _NEG_BIG, jnp.float32)
        ib_ref[...] = jnp.zeros((nrows, _SUB), jnp.int32)

    rowlane = rl_ref[...]

    def make_sub(masked):
        def sub(s, carry):
            vb, ib = carry
            base = j * chunk + s * _SUB
            x = x_ref[:, pl.ds(s * _SUB, _SUB)]
            i = rowlane + base.astype(jnp.uint32)
            g = _gumbel_from_bits(_threefry_bits(i))
            v = x + g
            if masked:
                lane = lax.broadcasted_iota(jnp.int32, (nrows, _SUB), 1)
                v = jnp.where(lane + base < ncols, v, _NEG_BIG)
            ib = jnp.where(v > vb, base, ib)
            vb = jnp.maximum(vb, v)
            return vb, ib

        return sub

    carry0 = (vb_ref[...], ib_ref[...])

    @pl.when(j < grid - 1)
    def _full():
        vb, ib = lax.fori_loop(0, nsub, make_sub(False), carry0, unroll=8)
        vb_ref[...] = vb
        ib_ref[...] = ib

    @pl.when(j == grid - 1)
    def _ragged():
        vb, ib = lax.fori_loop(0, nsub, make_sub(True), carry0, unroll=8)
        lane = lax.broadcasted_iota(jnp.int32, (nrows, _SUB), 1)
        m = jnp.max(vb, axis=1, keepdims=True)
        cand = jnp.where(vb == m, ib + lane, jnp.int32(ncols))
        o_ref[...] = jnp.min(cand, axis=1, keepdims=True)


@jax.jit
def kernel(outputs):
    nrows, ncols = outputs.shape
    chunk = 32768
    grid = pl.cdiv(ncols, chunk)
    rowlane = (
        jnp.arange(nrows, dtype=jnp.uint32)[:, None] * jnp.uint32(ncols)
        + jnp.arange(_SUB, dtype=jnp.uint32)[None, :]
    )

    out = pl.pallas_call(
        functools.partial(_body, ncols=ncols, chunk=chunk, nrows=nrows, grid=grid),
        grid=(grid,),
        in_specs=[
            pl.BlockSpec((nrows, chunk), lambda j: (0, j)),
            pl.BlockSpec((nrows, _SUB), lambda j: (0, 0)),
        ],
        out_specs=pl.BlockSpec((nrows, 1), lambda j: (0, 0)),
        out_shape=jax.ShapeDtypeStruct((nrows, 1), jnp.int32),
        scratch_shapes=[
            pltpu.VMEM((nrows, _SUB), jnp.float32),
            pltpu.VMEM((nrows, _SUB), jnp.int32),
        ],
        compiler_params=pltpu.CompilerParams(
            dimension_semantics=("arbitrary",),
        ),
    )(outputs, rowlane)
    return out


# speedup vs baseline: 1.0022x; 1.0022x over previous
import functools

import jax
import jax.numpy as jnp
from jax import lax
from jax.experimental import pallas as pl
from jax.experimental.pallas import tpu as pltpu

_ROT0 = (13, 15, 26, 6)
_ROT1 = (17, 29, 16, 24)
_TINY = float.fromhex("0x1p-126")
_NEG_BIG = -3.0e38
_SUB = 256


def _log2(x):
    return jnp.log2(x)


def _rotl(x, d):
    return lax.shift_left(x, jnp.uint32(d)) | lax.shift_right_logical(
        x, jnp.uint32(32 - d)
    )


def _threefry_bits(i):
    ks = (0, 42, 42 ^ 0x1BD11BDA)
    x1 = i + jnp.uint32(42)
    x0 = x1
    x1 = _rotl(x1, _ROT0[0]) ^ x0
    for r in _ROT0[1:]:
        x0 = x0 + x1
        x1 = _rotl(x1, r)
        x1 = x1 ^ x0
    x0 = x0 + jnp.uint32(ks[1] & 0xFFFFFFFF)
    x1 = x1 + jnp.uint32((ks[2] + 1) & 0xFFFFFFFF)
    for g in range(1, 5):
        rots = _ROT0 if g % 2 == 0 else _ROT1
        for r in rots:
            x0 = x0 + x1
            x1 = _rotl(x1, r)
            x1 = x1 ^ x0
        x0 = x0 + jnp.uint32(ks[(g + 1) % 3] & 0xFFFFFFFF)
        x1 = x1 + jnp.uint32((ks[(g + 2) % 3] + g + 1) & 0xFFFFFFFF)
    return x0 ^ x1


def _gumbel_from_bits(bits):
    fb = lax.shift_right_logical(bits, jnp.uint32(9)) | jnp.uint32(0x3F800000)
    u = lax.bitcast_convert_type(fb, jnp.float32) - jnp.float32(1.0)
    uu = u + _TINY
    nln2 = jnp.float32(-0.6931471805599453)
    m1 = _log2(uu) * nln2
    return _log2(m1) * nln2


def _body(x_ref, rl_ref, o_ref, vb_ref, ib_ref, *, ncols, chunk, nrows, grid):
    j = pl.program_id(0)
    nsub = chunk // _SUB

    @pl.when(j == 0)
    def _init():
        vb_ref[...] = jnp.full((nrows, _SUB), _NEG_BIG, jnp.float32)
        ib_ref[...] = jnp.zeros((nrows, _SUB), jnp.int32)

    rowlane = rl_ref[...]

    def make_sub(masked):
        def sub(s, carry):
            vb, ib = carry
            base = j * chunk + s * _SUB
            x = x_ref[:, pl.ds(s * _SUB, _SUB)]
            i = rowlane + base.astype(jnp.uint32)
            g = _gumbel_from_bits(_threefry_bits(i))
            v = x + g
            if masked:
                lane = lax.broadcasted_iota(jnp.int32, (nrows, _SUB), 1)
                v = jnp.where(lane + base < ncols, v, _NEG_BIG)
            ib = jnp.where(v > vb, base, ib)
            vb = jnp.maximum(vb, v)
            return vb, ib

        return sub

    carry0 = (vb_ref[...], ib_ref[...])

    @pl.when(j < grid - 1)
    def _full():
        vb, ib = lax.fori_loop(0, nsub, make_sub(False), carry0, unroll=16)
        vb_ref[...] = vb
        ib_ref[...] = ib

    @pl.when(j == grid - 1)
    def _ragged():
        vb, ib = lax.fori_loop(0, nsub, make_sub(True), carry0, unroll=16)
        lane = lax.broadcasted_iota(jnp.int32, (nrows, _SUB), 1)
        m = jnp.max(vb, axis=1, keepdims=True)
        cand = jnp.where(vb == m, ib + lane, jnp.int32(ncols))
        o_ref[...] = jnp.min(cand, axis=1, keepdims=True)


@jax.jit
def kernel(outputs):
    nrows, ncols = outputs.shape
    chunk = 32768
    grid = pl.cdiv(ncols, chunk)
    rowlane = (
        jnp.arange(nrows, dtype=jnp.uint32)[:, None] * jnp.uint32(ncols)
        + jnp.arange(_SUB, dtype=jnp.uint32)[None, :]
    )

    out = pl.pallas_call(
        functools.partial(_body, ncols=ncols, chunk=chunk, nrows=nrows, grid=grid),
        grid=(grid,),
        in_specs=[
            pl.BlockSpec((nrows, chunk), lambda j: (0, j)),
            pl.BlockSpec((nrows, _SUB), lambda j: (0, 0)),
        ],
        out_specs=pl.BlockSpec((nrows, 1), lambda j: (0, 0)),
        out_shape=jax.ShapeDtypeStruct((nrows, 1), jnp.int32),
        scratch_shapes=[
            pltpu.VMEM((nrows, _SUB), jnp.float32),
            pltpu.VMEM((nrows, _SUB), jnp.int32),
        ],
        compiler_params=pltpu.CompilerParams(
            dimension_semantics=("arbitrary",),
        ),
    )(outputs, rowlane)
    return out
